# baseline (speedup 1.0000x reference)
import jax
import jax.numpy as jnp
from jax import lax
from jax.experimental import pallas as pl
from jax.experimental.pallas import tpu as pltpu
from jax.experimental.pallas import tpu_sc as plsc

_M = 100000
_D = 3
_DP = 4
_B = 16384
_K = 64
_L = 16

_NC = 2
_NS = 16
_NW = _NC * _NS
_ROWS_W = _B // _NW
_CB = 32
_NCHUNK = _ROWS_W // _CB
_IDX_CHUNK = _CB * _K
_GDMA = 128
_NG = _IDX_CHUNK // _GDMA
_BPG = _GDMA // _K
_OUT_CHUNK = _IDX_CHUNK * _D
_OUT_VROWS = _OUT_CHUNK // _L
_VEC_B = (_K * _D) // _L


def _sc_body(idx_hbm, ptile_hbm, table_hbm, out_hbm, *scratch):
    idxb = scratch[:_NG]
    rowb = scratch[_NG:2 * _NG]
    ptile_v = scratch[2 * _NG]
    out_v = scratch[2 * _NG + 1]
    sem = scratch[2 * _NG + 2]

    wid = lax.axis_index("s") * _NC + lax.axis_index("c")

    lane = lax.iota(jnp.int32, _L)
    pr = [(_L * j + lane) // _D for j in range(_D)]
    pc = [(_L * j + lane) % _D for j in range(_D)]

    def chunk_body(c, carry):
        irow = wid * (_NCHUNK * _NG) + c * _NG
        for j in range(_NG):
            pltpu.sync_copy(idx_hbm.at[pl.ds(irow + j, 1)], idxb[j])
        copies = [
            pltpu.async_copy(table_hbm.at[idxb[j].at[0]], rowb[j], sem)
            for j in range(_NG)
        ]
        b0 = wid * _ROWS_W + c * _CB
        pltpu.sync_copy(ptile_hbm.at[pl.ds(b0, _CB)], ptile_v)
        for cp in copies:
            cp.wait()

        for j in range(_NG):
            for bb in range(_BPG):
                bc = j * _BPG + bb
                ts = [ptile_v[bc, pl.ds(_L * t, _L)] for t in range(_D)]
                for v in range(_VEC_B):
                    t = v % _D
                    roff = bb * _K + (v // _D) * _L
                    x = plsc.load_gather(rowb[j], [pr[t] + roff, pc[t]])
                    out_v[bc * _VEC_B + v, :] = x - ts[t]

        pltpu.sync_copy(out_v,
                        out_hbm.at[pl.ds((wid * _NCHUNK + c) * _OUT_VROWS,
                                         _OUT_VROWS)])
        return carry

    lax.fori_loop(0, _NCHUNK, chunk_body, 0)


def kernel(indices, points, sampleLocs):
    idx32 = indices.reshape(_B * _K).astype(jnp.int32)
    cooked = jnp.stack([idx32 * 2, jnp.zeros_like(idx32)], axis=-1)
    idxc = cooked.reshape(_B * _K // _GDMA, 2 * _GDMA)
    ptile = jnp.tile(points, (1, _L))
    tab4 = jnp.pad(sampleLocs, ((0, 0), (0, _DP - _D)))
    mesh = plsc.VectorSubcoreMesh(core_axis_name="c", subcore_axis_name="s")
    out_flat = pl.kernel(
        _sc_body,
        mesh=mesh,
        out_type=jax.ShapeDtypeStruct((_B * _K * _D // _L, _L), jnp.float32),
        scratch_types=(
            [pltpu.VMEM((1, 2 * _GDMA), jnp.int32) for _ in range(_NG)]
            + [pltpu.VMEM((2 * _GDMA, _DP), jnp.float32) for _ in range(_NG)]
            + [pltpu.VMEM((_CB, _D * _L), jnp.float32),
               pltpu.VMEM((_OUT_VROWS, _L), jnp.float32),
               pltpu.SemaphoreType.DMA]
        ),
        compiler_params=pltpu.CompilerParams(needs_layout_passes=False,
                                             use_tc_tiling_on_sc=False),
    )(idxc, ptile, tab4)
    return out_flat.reshape(_B, _K, _D)


# speedup vs baseline: 5.2143x; 5.2143x over previous
import jax
import jax.numpy as jnp
from jax import lax
from jax.experimental import pallas as pl
from jax.experimental.pallas import tpu as pltpu
from jax.experimental.pallas import tpu_sc as plsc

_M = 100000
_D = 3
_DP = 4
_B = 16384
_K = 64
_L = 16

_NC = 2
_NS = 16
_NW = _NC * _NS
_Q = _B * _K
_QW = _Q // _NW
_QC = 8192
_NOC = _QW // _QC
_TBR = 20480
_NBLK = 5
_MP = _TBR * _NBLK
_ROWS_W = _B // _NW
_CBO = _QC // _K
_VROWS = _QC * _D // _L
_VEC_B = (_K * _D) // _L


def _sc_body(idx_hbm, ptile_hbm, table_hbm, out_hbm,
             idxq, ptile_v, tabblk, out_v, sem):
    wid = lax.axis_index("s") * _NC + lax.axis_index("c")

    lane = lax.iota(jnp.int32, _L)
    pq = [(_L * j + lane) // _D for j in range(_D)]
    pc = [(_L * j + lane) % _D for j in range(_D)]

    for oc in range(_NOC):
        pltpu.sync_copy(idx_hbm.at[wid * _NOC + oc], idxq)
        b0 = wid * _ROWS_W + oc * _CBO
        pltpu.sync_copy(ptile_hbm.at[pl.ds(b0, _CBO)], ptile_v)

        for blk in range(_NBLK):
            pltpu.sync_copy(table_hbm.at[pl.ds(blk * (_TBR // 4), _TBR // 4)],
                            tabblk)
            base = blk * _TBR

            def vg_body(g, carry, base=base):
                qoff = g * _L
                for k in range(_D):
                    q = plsc.load_gather(idxq, [pq[k] + qoff])
                    ql = q - base
                    m = (ql >= 0) & (ql < _TBR)
                    qc = jnp.minimum(jnp.maximum(ql, 0), _TBR - 1)
                    w = qc * _DP + pc[k]
                    x = plsc.load_gather(tabblk, [w >> 4, w & 15], mask=m)
                    v = g * _D + k
                    out_v[v, :] = jnp.where(m, x, out_v[v, :])
                return carry

            lax.fori_loop(0, _VROWS // _D, vg_body, 0)

        def sub_body(b2, carry):
            ts = [ptile_v[b2, pl.ds(_L * t, _L)] for t in range(_D)]
            for v in range(_VEC_B):
                r = b2 * _VEC_B + v
                out_v[r, :] = out_v[r, :] - ts[v % _D]
            return carry

        lax.fori_loop(0, _CBO, sub_body, 0)
        pltpu.sync_copy(out_v,
                        out_hbm.at[pl.ds((wid * _NOC + oc) * _VROWS, _VROWS)])


def kernel(indices, points, sampleLocs):
    idxf = indices.reshape(_Q // _QC, _QC).astype(jnp.int32)
    ptile = jnp.tile(points, (1, _L))
    tabp = jnp.pad(sampleLocs, ((0, _MP - _M), (0, _DP - _D)))
    tabp = tabp.reshape(_MP // 4, 4 * _DP)
    mesh = plsc.VectorSubcoreMesh(core_axis_name="c", subcore_axis_name="s")
    out_flat = pl.kernel(
        _sc_body,
        mesh=mesh,
        out_type=jax.ShapeDtypeStruct((_Q * _D // _L, _L), jnp.float32),
        scratch_types=[
            pltpu.VMEM((_QC,), jnp.int32),
            pltpu.VMEM((_CBO, _D * _L), jnp.float32),
            pltpu.VMEM((_TBR // 4, 4 * _DP), jnp.float32),
            pltpu.VMEM((_VROWS, _L), jnp.float32),
            pltpu.SemaphoreType.DMA,
        ],
        compiler_params=pltpu.CompilerParams(needs_layout_passes=False,
                                             use_tc_tiling_on_sc=False),
    )(idxf, ptile, tabp)
    return out_flat.reshape(_B, _K, _D)


# speedup vs baseline: 5.7966x; 1.1117x over previous
import jax
import jax.numpy as jnp
from jax import lax
from jax.experimental import pallas as pl
from jax.experimental.pallas import tpu as pltpu
from jax.experimental.pallas import tpu_sc as plsc

_M = 100000
_D = 3
_B = 16384
_K = 64
_L = 16

_NC = 2
_NS = 16
_NW = _NC * _NS
_Q = _B * _K
_QW = _Q // _NW
_QC = 8192
_NOC = _QW // _QC
_MW = _M * _D // _L
_TBW = 5120
_NBLK = (_MW + _TBW - 1) // _TBW
_ROWS_W = _B // _NW
_CBO = _QC // _K
_VROWS = _QC * _D // _L
_VEC_B = (_K * _D) // _L


def _sc_body(idx_hbm, pts_hbm, table_hbm, out_hbm,
             idxq, pts_v, tabblk, out_v, sem):
    wid = lax.axis_index("s") * _NC + lax.axis_index("c")

    lane = lax.iota(jnp.int32, _L)
    pq = [(_L * j + lane) // _D for j in range(_D)]
    pc = [(_L * j + lane) % _D for j in range(_D)]

    for oc in range(_NOC):
        pltpu.sync_copy(idx_hbm.at[wid * _NOC + oc], idxq)
        b0 = wid * _ROWS_W + oc * _CBO
        pltpu.sync_copy(pts_hbm.at[pl.ds(b0, _CBO)], pts_v)

        for blk in range(_NBLK):
            bw = min(_TBW, _MW - blk * _TBW)
            pltpu.sync_copy(table_hbm.at[pl.ds(blk * _TBW, bw)],
                            tabblk.at[pl.ds(0, bw)])
            nwords = bw * _L
            pcb = [pc[j] - (blk * _TBW * _L) for j in range(_D)]

            def vg_body(g, carry, pcb=pcb, nwords=nwords):
                qoff = g * _L
                for k in range(_D):
                    q = plsc.load_gather(idxq, [pq[k] + qoff])
                    w = q * _D + pcb[k]
                    m = (w >= 0) & (w < nwords)
                    wc = jnp.minimum(jnp.maximum(w, 0), nwords - 1)
                    x = plsc.load_gather(tabblk, [wc >> 4, wc & 15], mask=m)
                    v = g * _D + k
                    out_v[v, :] = jnp.where(m, x, out_v[v, :])
                return carry

            lax.fori_loop(0, _VROWS // _D, vg_body, 0)

        def sub_body(b2, carry):
            ts = [plsc.load_gather(pts_v, [jnp.zeros((_L,), jnp.int32) + b2,
                                           pc[t]]) for t in range(_D)]
            for v in range(_VEC_B):
                r = b2 * _VEC_B + v
                out_v[r, :] = out_v[r, :] - ts[v % _D]
            return carry

        lax.fori_loop(0, _CBO, sub_body, 0)
        pltpu.sync_copy(out_v,
                        out_hbm.at[pl.ds((wid * _NOC + oc) * _VROWS, _VROWS)])


def kernel(indices, points, sampleLocs):
    idxf = indices.reshape(_Q // _QC, _QC).astype(jnp.int32)
    tabw = sampleLocs.reshape(_MW, _L)
    mesh = plsc.VectorSubcoreMesh(core_axis_name="c", subcore_axis_name="s")
    out_flat = pl.kernel(
        _sc_body,
        mesh=mesh,
        out_type=jax.ShapeDtypeStruct((_Q * _D // _L, _L), jnp.float32),
        scratch_types=[
            pltpu.VMEM((_QC,), jnp.int32),
            pltpu.VMEM((_CBO, _D), jnp.float32),
            pltpu.VMEM((_TBW, _L), jnp.float32),
            pltpu.VMEM((_VROWS, _L), jnp.float32),
            pltpu.SemaphoreType.DMA,
        ],
        compiler_params=pltpu.CompilerParams(needs_layout_passes=False,
                                             use_tc_tiling_on_sc=False),
    )(idxf, points, tabw)
    return out_flat.reshape(_B, _K, _D)


# speedup vs baseline: 5.8389x; 1.0073x over previous
import jax
import jax.numpy as jnp
from jax import lax
from jax.experimental import pallas as pl
from jax.experimental.pallas import tpu as pltpu
from jax.experimental.pallas import tpu_sc as plsc

_M = 100000
_D = 3
_B = 16384
_K = 64
_L = 16

_NC = 2
_NS = 16
_NW = _NC * _NS
_Q = _B * _K
_QW = _Q // _NW
_QC = 8192
_NOC = _QW // _QC
_MW = _M * _D // _L
_TBW = 5120
_NBLK = (_MW + _TBW - 1) // _TBW
_ROWS_W = _B // _NW
_CBO = _QC // _K
_VROWS = _QC * _D // _L
_VEC_B = (_K * _D) // _L


def _sc_body(idx_hbm, pts_hbm, table_hbm, out_hbm,
             idxq, pts_v, tabblk, out_v, sem):
    wid = lax.axis_index("s") * _NC + lax.axis_index("c")

    lane = lax.iota(jnp.int32, _L)
    pq = [(_L * j + lane) // _D for j in range(_D)]
    pc = [(_L * j + lane) % _D for j in range(_D)]

    for oc in range(_NOC):
        pltpu.sync_copy(idx_hbm.at[wid * _NOC + oc], idxq)
        b0 = wid * _ROWS_W + oc * _CBO
        pltpu.sync_copy(pts_hbm.at[pl.ds(b0, _CBO)], pts_v)

        for blk in range(_NBLK):
            bw = min(_TBW, _MW - blk * _TBW)
            pltpu.sync_copy(table_hbm.at[pl.ds(blk * _TBW, bw)],
                            tabblk.at[pl.ds(0, bw)])
            nwords = bw * _L
            pcb = [pc[j] - (blk * _TBW * _L) for j in range(_D)]

            def vg_body(g, carry, pcb=pcb, nwords=nwords):
                qoff = g * _L
                for k in range(_D):
                    q = plsc.load_gather(idxq, [pq[k] + qoff])
                    w = q * _D + pcb[k]
                    wu = plsc.bitcast(w, jnp.uint32)
                    m = wu < nwords
                    wc = plsc.bitcast(jnp.minimum(wu, nwords - 1), jnp.int32)
                    x = plsc.load_gather(tabblk, [wc >> 4, wc & 15], mask=m)
                    v = g * _D + k
                    out_v[v, :] = jnp.where(m, x, out_v[v, :])
                return carry

            lax.fori_loop(0, _VROWS // _D, vg_body, 0)

        def sub_body(b2, carry):
            ts = [plsc.load_gather(pts_v, [jnp.zeros((_L,), jnp.int32) + b2,
                                           pc[t]]) for t in range(_D)]
            for v in range(_VEC_B):
                r = b2 * _VEC_B + v
                out_v[r, :] = out_v[r, :] - ts[v % _D]
            return carry

        lax.fori_loop(0, _CBO, sub_body, 0)
        pltpu.sync_copy(out_v,
                        out_hbm.at[pl.ds((wid * _NOC + oc) * _VROWS, _VROWS)])


def kernel(indices, points, sampleLocs):
    idxf = indices.reshape(_Q // _QC, _QC).astype(jnp.int32)
    tabw = sampleLocs.reshape(_MW, _L)
    mesh = plsc.VectorSubcoreMesh(core_axis_name="c", subcore_axis_name="s")
    out_flat = pl.kernel(
        _sc_body,
        mesh=mesh,
        out_type=jax.ShapeDtypeStruct((_Q * _D // _L, _L), jnp.float32),
        scratch_types=[
            pltpu.VMEM((_QC,), jnp.int32),
            pltpu.VMEM((_CBO, _D), jnp.float32),
            pltpu.VMEM((_TBW, _L), jnp.float32),
            pltpu.VMEM((_VROWS, _L), jnp.float32),
            pltpu.SemaphoreType.DMA,
        ],
        compiler_params=pltpu.CompilerParams(needs_layout_passes=False,
                                             use_tc_tiling_on_sc=False),
    )(idxf, points, tabw)
    zero = jnp.min(jnp.abs(points)) * jnp.float32(0.0)
    return out_flat.reshape(_B, _K, _D) + zero


# speedup vs baseline: 5.9226x; 1.0143x over previous
import jax
import jax.numpy as jnp
from jax import lax
from jax.experimental import pallas as pl
from jax.experimental.pallas import tpu as pltpu
from jax.experimental.pallas import tpu_sc as plsc

_M = 100000
_D = 3
_B = 16384
_K = 64
_L = 16

_NC = 2
_NS = 16
_NW = _NC * _NS
_Q = _B * _K
_QW = _Q // _NW
_QC = 8192
_NOC = _QW // _QC
_MW = _M * _D // _L
_TBW = 5120
_NBLK = (_MW + _TBW - 1) // _TBW
_ROWS_W = _B // _NW
_CBO = _QC // _K
_VROWS = _QC * _D // _L
_VEC_B = (_K * _D) // _L
_UNROLL = 4


def _sc_body(idx_hbm, pts_hbm, table_hbm, out_hbm,
             idxq, pts_v, tabblk, out_v, sem):
    wid = lax.axis_index("s") * _NC + lax.axis_index("c")

    lane = lax.iota(jnp.int32, _L)
    pq = [(_L * j + lane) // _D for j in range(_D)]
    pc = [(_L * j + lane) % _D for j in range(_D)]

    for oc in range(_NOC):
        pltpu.sync_copy(idx_hbm.at[wid * _NOC + oc], idxq)
        b0 = wid * _ROWS_W + oc * _CBO
        pltpu.sync_copy(pts_hbm.at[pl.ds(b0, _CBO)], pts_v)

        for blk in range(_NBLK):
            bw = min(_TBW, _MW - blk * _TBW)
            pltpu.sync_copy(table_hbm.at[pl.ds(blk * _TBW, bw)],
                            tabblk.at[pl.ds(0, bw)])
            nwords = bw * _L
            pcb = [pc[j] - (blk * _TBW * _L) for j in range(_D)]

            def vg_body(gg, carry, pcb=pcb, nwords=nwords):
                for u in range(_UNROLL):
                    g = gg * _UNROLL + u
                    qoff = g * _L
                    for k in range(_D):
                        q = plsc.load_gather(idxq, [pq[k] + qoff])
                        w = q * _D + pcb[k]
                        wu = plsc.bitcast(w, jnp.uint32)
                        m = wu < nwords
                        wc = plsc.bitcast(jnp.minimum(wu, nwords - 1),
                                          jnp.int32)
                        x = plsc.load_gather(tabblk, [wc >> 4, wc & 15],
                                             mask=m)
                        v = g * _D + k
                        out_v[v, :] = jnp.where(m, x, out_v[v, :])
                return carry

            lax.fori_loop(0, _VROWS // (_D * _UNROLL), vg_body, 0)

        def sub_body(b2, carry):
            ts = [plsc.load_gather(pts_v, [jnp.zeros((_L,), jnp.int32) + b2,
                                           pc[t]]) for t in range(_D)]
            for v in range(_VEC_B):
                r = b2 * _VEC_B + v
                out_v[r, :] = out_v[r, :] - ts[v % _D]
            return carry

        lax.fori_loop(0, _CBO, sub_body, 0)
        pltpu.sync_copy(out_v,
                        out_hbm.at[pl.ds((wid * _NOC + oc) * _VROWS, _VROWS)])


def kernel(indices, points, sampleLocs):
    idxf = indices.reshape(_Q // _QC, _QC).astype(jnp.int32)
    tabw = sampleLocs.reshape(_MW, _L)
    mesh = plsc.VectorSubcoreMesh(core_axis_name="c", subcore_axis_name="s")
    out_flat = pl.kernel(
        _sc_body,
        mesh=mesh,
        out_type=jax.ShapeDtypeStruct((_Q * _D // _L, _L), jnp.float32),
        scratch_types=[
            pltpu.VMEM((_QC,), jnp.int32),
            pltpu.VMEM((_CBO, _D), jnp.float32),
            pltpu.VMEM((_TBW, _L), jnp.float32),
            pltpu.VMEM((_VROWS, _L), jnp.float32),
            pltpu.SemaphoreType.DMA,
        ],
        compiler_params=pltpu.CompilerParams(needs_layout_passes=False,
                                             use_tc_tiling_on_sc=False),
    )(idxf, points, tabw)
    zero = jnp.min(jnp.abs(points)) * jnp.float32(0.0)
    return out_flat.reshape(_B, _K, _D) + zero


# speedup vs baseline: 5.9536x; 1.0052x over previous
import jax
import jax.numpy as jnp
from jax import lax
from jax.experimental import pallas as pl
from jax.experimental.pallas import tpu as pltpu
from jax.experimental.pallas import tpu_sc as plsc

_M = 100000
_D = 3
_B = 16384
_K = 64
_L = 16

_NC = 2
_NS = 16
_NW = _NC * _NS
_Q = _B * _K
_QW = _Q // _NW
_QC = 8192
_NOC = _QW // _QC
_MW = _M * _D // _L
_TBW = 5120
_NBLK = (_MW + _TBW - 1) // _TBW
_ROWS_W = _B // _NW
_CBO = _QC // _K
_VROWS = _QC * _D // _L
_VEC_B = (_K * _D) // _L
_UNROLL = 4


def _sc_body(idx_hbm, pts_hbm, table_hbm, out_hbm,
             idxq, pts_v, tabblk, out_v, sem):
    wid = lax.axis_index("s") * _NC + lax.axis_index("c")

    lane = lax.iota(jnp.int32, _L)
    pq = [(_L * j + lane) // _D for j in range(_D)]
    pc = [(_L * j + lane) % _D for j in range(_D)]

    for oc in range(_NOC):
        pltpu.sync_copy(idx_hbm.at[wid * _NOC + oc], idxq)
        b0 = wid * _ROWS_W + oc * _CBO
        pltpu.sync_copy(pts_hbm.at[pl.ds(b0, _CBO)], pts_v)

        for blk in range(_NBLK):
            bw = min(_TBW, _MW - blk * _TBW)
            pltpu.sync_copy(table_hbm.at[pl.ds(blk * _TBW, bw)],
                            tabblk.at[pl.ds(0, bw)])
            nwords = bw * _L
            pcb = [pc[j] - (blk * _TBW * _L) for j in range(_D)]

            def vg_body(gg, carry, pcb=pcb, nwords=nwords):
                for u in range(_UNROLL):
                    g = gg * _UNROLL + u
                    qoff = g * _L
                    for k in range(_D):
                        q = plsc.load_gather(idxq, [pq[k] + qoff])
                        w = q * _D + pcb[k]
                        wu = plsc.bitcast(w, jnp.uint32)
                        m = wu < nwords
                        wc = plsc.bitcast(jnp.minimum(wu, nwords - 1),
                                          jnp.int32)
                        x = plsc.load_gather(tabblk, [wc >> 4, wc & 15],
                                             mask=m)
                        vo = (g * _D + k) * _L
                        out_v[pl.ds(vo, _L)] = jnp.where(
                            m, x, out_v[pl.ds(vo, _L)])
                return carry

            lax.fori_loop(0, _VROWS // (_D * _UNROLL), vg_body, 0)

        def sub_body(b2, carry):
            ts = [plsc.load_gather(pts_v, [jnp.zeros((_L,), jnp.int32) + b2,
                                           pc[t]]) for t in range(_D)]
            for v in range(_VEC_B):
                r = (b2 * _VEC_B + v) * _L
                out_v[pl.ds(r, _L)] = out_v[pl.ds(r, _L)] - ts[v % _D]
            return carry

        lax.fori_loop(0, _CBO, sub_body, 0)
        pltpu.sync_copy(out_v,
                        out_hbm.at[pl.ds((wid * _NOC + oc) * _VROWS * _L,
                                         _VROWS * _L)])


def kernel(indices, points, sampleLocs):
    idxf = indices.reshape(_Q // _QC, _QC).astype(jnp.int32)
    tabw = sampleLocs.reshape(_MW, _L)
    mesh = plsc.VectorSubcoreMesh(core_axis_name="c", subcore_axis_name="s")
    out_flat = pl.kernel(
        _sc_body,
        mesh=mesh,
        out_type=jax.ShapeDtypeStruct((_Q * _D,), jnp.float32),
        scratch_types=[
            pltpu.VMEM((_QC,), jnp.int32),
            pltpu.VMEM((_CBO, _D), jnp.float32),
            pltpu.VMEM((_TBW, _L), jnp.float32),
            pltpu.VMEM((_VROWS * _L,), jnp.float32),
            pltpu.SemaphoreType.DMA,
        ],
        compiler_params=pltpu.CompilerParams(needs_layout_passes=False,
                                             use_tc_tiling_on_sc=False),
    )(idxf, points, tabw)
    return out_flat.reshape(_B, _K, _D)
